# baseline (speedup 1.0000x reference)
import jax
import jax.numpy as jnp
from jax.experimental import pallas as pl
from jax.experimental.pallas import tpu as pltpu

_N_CHUNKS = 8


def _copy_body(x_ref, o_ref, sems):
    copies = []
    for i in range(_N_CHUNKS):
        c = pltpu.make_async_copy(x_ref.at[i], o_ref.at[i], sems.at[i])
        c.start()
        copies.append(c)
    for c in copies:
        c.wait()


def kernel(x, pe_weight):
    del pe_weight
    b, s, d = x.shape
    rows = b * s
    x2 = x.reshape(_N_CHUNKS, rows // _N_CHUNKS, d)
    out = pl.pallas_call(
        _copy_body,
        out_shape=jax.ShapeDtypeStruct(x2.shape, x.dtype),
        in_specs=[pl.BlockSpec(memory_space=pl.ANY)],
        out_specs=pl.BlockSpec(memory_space=pl.ANY),
        scratch_shapes=[pltpu.SemaphoreType.DMA((_N_CHUNKS,))],
    )(x2)
    return out.reshape(b, s, d)


# speedup vs baseline: 31.0992x; 31.0992x over previous
import jax
import jax.numpy as jnp
from jax.experimental import pallas as pl
from jax.experimental.pallas import tpu as pltpu


def _copy_body(x_ref, o_ref):
    o_ref[...] = x_ref[...]


def kernel(x, pe_weight):
    del pe_weight
    b, s, d = x.shape
    x2 = x.reshape(b * s, d)
    rows = b * s
    block_rows = 256
    out = pl.pallas_call(
        _copy_body,
        out_shape=jax.ShapeDtypeStruct((rows, d), x.dtype),
        grid=(rows // block_rows,),
        in_specs=[pl.BlockSpec((block_rows, d), lambda i: (i, 0))],
        out_specs=pl.BlockSpec((block_rows, d), lambda i: (i, 0)),
        compiler_params=pltpu.CompilerParams(
            dimension_semantics=("arbitrary",),
        ),
    )(x2)
    return out.reshape(b, s, d)


# speedup vs baseline: 34.0050x; 1.0934x over previous
import jax
import jax.numpy as jnp
from jax.experimental import pallas as pl
from jax.experimental.pallas import tpu as pltpu

_N_CHUNKS = 32


def _copy_body(x_hbm, o_hbm, buf, in_sems, out_sems):
    i = pl.program_id(0)
    n = pl.num_programs(0)
    slot = jax.lax.rem(i, 2)
    nslot = jax.lax.rem(i + 1, 2)

    def in_copy(c, s):
        return pltpu.make_async_copy(x_hbm.at[c], buf.at[s], in_sems.at[s])

    def out_copy(c, s):
        return pltpu.make_async_copy(buf.at[s], o_hbm.at[c], out_sems.at[s])

    @pl.when(i == 0)
    def _():
        in_copy(0, 0).start()

    @pl.when(jax.lax.bitwise_and(i >= 1, i + 1 < n))
    def _():
        out_copy(i - 1, nslot).wait()

    @pl.when(i + 1 < n)
    def _():
        in_copy(i + 1, nslot).start()

    in_copy(i, slot).wait()
    out_copy(i, slot).start()

    @pl.when(i == n - 1)
    def _():
        out_copy(i - 1, nslot).wait()
        out_copy(i, slot).wait()


def kernel(x, pe_weight):
    del pe_weight
    b, s, d = x.shape
    rows = b * s
    x2 = x.reshape(_N_CHUNKS, rows // _N_CHUNKS, d)
    out = pl.pallas_call(
        _copy_body,
        out_shape=jax.ShapeDtypeStruct(x2.shape, x.dtype),
        grid=(_N_CHUNKS,),
        in_specs=[pl.BlockSpec(memory_space=pl.ANY)],
        out_specs=pl.BlockSpec(memory_space=pl.ANY),
        scratch_shapes=[
            pltpu.VMEM((2, rows // _N_CHUNKS, d), x.dtype),
            pltpu.SemaphoreType.DMA((2,)),
            pltpu.SemaphoreType.DMA((2,)),
        ],
        compiler_params=pltpu.CompilerParams(
            dimension_semantics=("arbitrary",),
        ),
    )(x2)
    return out.reshape(b, s, d)
